# baseline (speedup 1.0000x reference)
import jax
import jax.numpy as jnp
from jax.experimental import pallas as pl
from jax.experimental.pallas import tpu as pltpu

_N = 4096
_D = 64
_NP = 4
_NA = 32
_H = 64
_TOPK = 32
_ZD = _NP * _D + _H
_BLOCK = 256


def _z_kernel(ctx_ref, pe_ref, w_ref, wpe_ref, ps_ref, z_ref):
    ctx = ctx_ref[...]
    w = w_ref[...]
    for p in range(_NP):
        x = ctx * w[p, :][None, :]
        nrm = jnp.sqrt(jnp.sum(x * x, axis=1, keepdims=True))
        x = x / jnp.maximum(nrm, 1e-12)
        z_ref[:, p * _D:(p + 1) * _D] = x
    pe = jax.lax.dot_general(
        pe_ref[...], wpe_ref[...], (((1,), (0,)), ((), ())),
        preferred_element_type=jnp.float32)
    z_ref[:, _NP * _D:] = pe * ps_ref[0, 0]


def _topk_kernel(zrow_ref, zall_ref, gpr_ref, os_ref,
                 out_ref, v_ref):
    zr = zrow_ref[...]
    za = zall_ref[...]
    dn = (((1,), (1,)), ((), ()))
    s = jax.lax.dot_general(
        zr, za, dn, preferred_element_type=jnp.float32)
    s = s * os_ref[0, 0]
    s = s * gpr_ref[...]
    out_ref[...] = s

    b = jax.lax.bitcast_convert_type(s, jnp.int32)
    key = b ^ jax.lax.shift_right_arithmetic(b, 31).__and__(0x7FFFFFFF)
    v_ref[...] = key
    lo = jnp.min(key, axis=1, keepdims=True)
    hi = jnp.max(key, axis=1, keepdims=True) + 1

    def cond(state):
        lo, hi = state
        return jnp.any(jax.lax.shift_right_logical(hi - lo, 1) != 0)

    def body(state):
        lo, hi = state
        half = jax.lax.shift_right_logical(hi - lo, 1)
        open_ = half != 0
        mid = lo + half
        k = v_ref[...]
        c = jnp.sum((k >= mid).astype(jnp.int32), axis=1, keepdims=True)
        ge = c >= _TOPK
        eq = c == _TOPK
        lo = jnp.where(jnp.logical_and(open_, ge), mid, lo)
        hi = jnp.where(
            jnp.logical_and(open_, jnp.logical_not(ge)), mid,
            jnp.where(jnp.logical_and(open_, eq), mid + 1, hi))
        return lo, hi

    lo, hi = jax.lax.while_loop(cond, body, (lo, hi))
    tb = lo ^ jax.lax.shift_right_arithmetic(lo, 31).__and__(0x7FFFFFFF)
    t = jax.lax.bitcast_convert_type(tb, jnp.float32)
    s = out_ref[...]
    out_ref[...] = jnp.where(s >= t, s, 0.0)


def kernel(context, position_encoding, gpr_rank, position_flag, W, Wpe):
    flag = jnp.asarray(position_flag)
    ps = jnp.where(flag == 1, 2.0, 0.0).astype(jnp.float32).reshape(1, 1)
    os_ = jnp.where(flag == 1, 0.125, 0.25).astype(jnp.float32).reshape(1, 1)

    z = pl.pallas_call(
        _z_kernel,
        out_shape=jax.ShapeDtypeStruct((_N, _ZD), jnp.float32),
    )(context, position_encoding, W, Wpe, ps)

    out = pl.pallas_call(
        _topk_kernel,
        grid=(_N // _BLOCK,),
        in_specs=[
            pl.BlockSpec((_BLOCK, _ZD), lambda i: (i, 0)),
            pl.BlockSpec((_N, _ZD), lambda i: (0, 0)),
            pl.BlockSpec((_BLOCK, 1), lambda i: (i, 0)),
            pl.BlockSpec((1, 1), lambda i: (0, 0)),
        ],
        out_specs=pl.BlockSpec((_BLOCK, _N), lambda i: (i, 0)),
        out_shape=jax.ShapeDtypeStruct((_N, _N), jnp.float32),
        scratch_shapes=[pltpu.VMEM((_BLOCK, _N), jnp.int32)],
        compiler_params=pltpu.CompilerParams(
            dimension_semantics=("arbitrary",)),
    )(z, z, gpr_rank, os_)
    return out


# speedup vs baseline: 1.2703x; 1.2703x over previous
import jax
import jax.numpy as jnp
from jax.experimental import pallas as pl
from jax.experimental.pallas import tpu as pltpu

_N = 4096
_D = 64
_NP = 4
_NA = 32
_H = 64
_TOPK = 32
_ZD = _NP * _D + _H
_BLOCK = 256
_CHUNK = 128


def _z_kernel(ctx_ref, pe_ref, w_ref, wpe_ref, ps_ref, z_ref):
    ctx = ctx_ref[...]
    w = w_ref[...]
    for p in range(_NP):
        x = ctx * w[p, :][None, :]
        nrm = jnp.sqrt(jnp.sum(x * x, axis=1, keepdims=True))
        x = x / jnp.maximum(nrm, 1e-12)
        z_ref[:, p * _D:(p + 1) * _D] = x
    pe = jax.lax.dot_general(
        pe_ref[...], wpe_ref[...], (((1,), (0,)), ((), ())),
        preferred_element_type=jnp.float32)
    z_ref[:, _NP * _D:] = pe * ps_ref[0, 0]


def _to_key(v):
    b = jax.lax.bitcast_convert_type(v, jnp.int32)
    return b ^ jax.lax.shift_right_arithmetic(b, 31).__and__(0x7FFFFFFF)


def _to_float(k):
    b = k ^ jax.lax.shift_right_arithmetic(k, 31).__and__(0x7FFFFFFF)
    return jax.lax.bitcast_convert_type(b, jnp.float32)


def _topk_kernel(zrow_ref, zall_ref, gpr_ref, os_ref, out_ref, v_ref):
    zr = zrow_ref[...]
    za = zall_ref[...]
    dn = (((1,), (1,)), ((), ()))
    st = jax.lax.dot_general(
        za, zr, dn, preferred_element_type=jnp.float32)
    st = st * os_ref[0, 0]
    st = st * gpr_ref[...]
    v_ref[...] = st

    m = jnp.max(st.reshape(_N // _CHUNK, _CHUNK, _BLOCK), axis=1)
    mk = _to_key(m)
    lo = jnp.min(mk, axis=0, keepdims=True)
    hi = jnp.max(mk, axis=0, keepdims=True) + 1

    def cond(state):
        lo, hi = state
        return jnp.any(jax.lax.shift_right_logical(hi - lo, 1) != 0)

    def body(state):
        lo, hi = state
        half = jax.lax.shift_right_logical(hi - lo, 1)
        open_ = half != 0
        mid = lo + half
        tmid = _to_float(mid)
        v = v_ref[...]
        c = jnp.sum((v >= tmid).astype(jnp.int32), axis=0, keepdims=True)
        ge = c >= _TOPK
        eq = c == _TOPK
        lo = jnp.where(jnp.logical_and(open_, ge), mid, lo)
        hi = jnp.where(
            jnp.logical_and(open_, jnp.logical_not(ge)), mid,
            jnp.where(jnp.logical_and(open_, eq), mid + 1, hi))
        return lo, hi

    lo, hi = jax.lax.while_loop(cond, body, (lo, hi))
    t = _to_float(lo)
    v = v_ref[...]
    out_ref[...] = jnp.where(v >= t, v, 0.0).T


def kernel(context, position_encoding, gpr_rank, position_flag, W, Wpe):
    flag = jnp.asarray(position_flag)
    ps = jnp.where(flag == 1, 2.0, 0.0).astype(jnp.float32).reshape(1, 1)
    os_ = jnp.where(flag == 1, 0.125, 0.25).astype(jnp.float32).reshape(1, 1)
    gpr_row = gpr_rank.reshape(1, _N)

    z = pl.pallas_call(
        _z_kernel,
        out_shape=jax.ShapeDtypeStruct((_N, _ZD), jnp.float32),
    )(context, position_encoding, W, Wpe, ps)

    out = pl.pallas_call(
        _topk_kernel,
        grid=(_N // _BLOCK,),
        in_specs=[
            pl.BlockSpec((_BLOCK, _ZD), lambda i: (i, 0)),
            pl.BlockSpec((_N, _ZD), lambda i: (0, 0)),
            pl.BlockSpec((1, _BLOCK), lambda i: (0, i)),
            pl.BlockSpec((1, 1), lambda i: (0, 0)),
        ],
        out_specs=pl.BlockSpec((_BLOCK, _N), lambda i: (i, 0)),
        out_shape=jax.ShapeDtypeStruct((_N, _N), jnp.float32),
        scratch_shapes=[pltpu.VMEM((_N, _BLOCK), jnp.float32)],
        compiler_params=pltpu.CompilerParams(
            dimension_semantics=("arbitrary",)),
    )(z, z, gpr_row, os_)
    return out
